# Initial kernel scaffold; baseline (speedup 1.0000x reference)
import jax
import jax.numpy as jnp
from jax.experimental import pallas as pl


def kernel(input, embed, bn_weight, bn_bias, bn_mean, bn_var):
    raise NotImplementedError("write your pallas kernel here")


# speedup vs baseline: 1.3185x; 1.3185x over previous
import jax
import jax.numpy as jnp
from jax.experimental import pallas as pl
from jax.experimental.pallas import tpu as pltpu
from jax.experimental.pallas import tpu_sc as plsc

_BN_EPS = 1e-05
_BM = 256


def _tc_body(x_ref, emb_ref, w_ref, b_ref, mu_ref, var_ref,
             idx_ref, dsum_ref, *, n_embed, n_blocks, inv_count):
    i = pl.program_id(0)
    x = x_ref[...]
    xn = (x - mu_ref[...]) * jax.lax.rsqrt(var_ref[...] + _BN_EPS) \
        * w_ref[...] + b_ref[...]
    emb = emb_ref[...]
    mm2 = 2.0 * jnp.dot(xn, emb, preferred_element_type=jnp.float32)
    rowsq = jnp.sum(xn * xn, axis=1, keepdims=True)
    colsq = jnp.sum(emb * emb, axis=0, keepdims=True)
    neg = (mm2 - rowsq) - colsq
    maxv = jnp.max(neg, axis=1, keepdims=True)
    iota = jax.lax.broadcasted_iota(jnp.int32, neg.shape, 1)
    idx = jnp.min(jnp.where(neg == maxv, iota, n_embed), axis=1)
    idx_ref[0, 0, :] = idx.astype(jnp.int32)
    part = jnp.sum(-maxv, axis=0, keepdims=True)
    prev = jnp.where(i == 0, jnp.zeros_like(part), dsum_ref[...])
    acc = prev + part
    dsum_ref[...] = jnp.where(i == n_blocks - 1, acc * inv_count, acc)


def _argmin_dist(x2d, embed, bn_weight, bn_bias, bn_mean, bn_var):
    m, d = x2d.shape
    n = embed.shape[1]
    nb = m // _BM
    from functools import partial
    body = partial(_tc_body, n_embed=n, n_blocks=nb,
                   inv_count=1.0 / float(m * d))
    idx, dsum = pl.pallas_call(
        body,
        grid=(nb,),
        in_specs=[
            pl.BlockSpec((_BM, d), lambda i: (i, 0)),
            pl.BlockSpec((d, n), lambda i: (0, 0)),
            pl.BlockSpec((1, d), lambda i: (0, 0)),
            pl.BlockSpec((1, d), lambda i: (0, 0)),
            pl.BlockSpec((1, d), lambda i: (0, 0)),
            pl.BlockSpec((1, d), lambda i: (0, 0)),
        ],
        out_specs=[
            pl.BlockSpec((1, 1, _BM), lambda i: (i, 0, 0)),
            pl.BlockSpec((1, 1), lambda i: (0, 0)),
        ],
        out_shape=[
            jax.ShapeDtypeStruct((nb, 1, _BM), jnp.int32),
            jax.ShapeDtypeStruct((1, 1), jnp.float32),
        ],
        compiler_params=pltpu.CompilerParams(
            dimension_semantics=("arbitrary",)),
    )(x2d, embed,
      bn_weight.reshape(1, d), bn_bias.reshape(1, d),
      bn_mean.reshape(1, d), bn_var.reshape(1, d))
    return idx.reshape(1, m), dsum[0, 0]


_GW = 128


def _sc_gather(table, indices):
    m = indices.shape[1]
    d = table.shape[1]
    mesh = plsc.VectorSubcoreMesh(core_axis_name="core",
                                  subcore_axis_name="subcore")

    @pl.kernel(out_type=jax.ShapeDtypeStruct((m, d), table.dtype), mesh=mesh)
    def k(tab_hbm, i_hbm, o_hbm):
        def body(i_vmem, o_vmem):
            pltpu.sync_copy(tab_hbm.at[i_vmem.at[0]], o_vmem)

        pltpu.emit_pipeline(
            body,
            grid=(m // _GW,),
            in_specs=[pl.BlockSpec((1, _GW), index_map=lambda i: (0, i))],
            out_specs=[pl.BlockSpec((_GW, d), index_map=lambda i: (i, 0))],
            core_axis_name=("core", "subcore"),
            dimension_semantics=(pltpu.PARALLEL,),
        )(i_hbm, o_hbm)

    return k(table, indices)


def kernel(input, embed, bn_weight, bn_bias, bn_mean, bn_var):
    d = input.shape[-1]
    m = input.size // d
    x2d = input.reshape(m, d)
    idx, diff = _argmin_dist(x2d, embed, bn_weight, bn_bias, bn_mean, bn_var)
    table = jnp.pad(embed.T, ((0, 0), (0, 128 - d)))
    quantize = _sc_gather(table, idx)[:, :d]
    return quantize.reshape(input.shape), diff
